# baseline (speedup 1.0000x reference)
import functools

import jax
import jax.numpy as jnp
from jax import lax
from jax.experimental import pallas as pl
from jax.experimental.pallas import tpu as pltpu
from jax.experimental.pallas import tpu_sc as plsc

B = 16384
E = 64
MF = 3
BINS = 50
FDIM = 73
FIN = 265
D = 1164

NC, NS = 2, 16
NW = NC * NS
CHUNK = 128

ROWS3 = 3 * B
R3_W = ROWS3 // NW

RBLK = 1024


_GDN = lax.GatherDimensionNumbers(
    offset_dims=(), collapsed_slice_dims=(0,), start_index_map=(0,))


def _bcast_lane(v, r):
    idx = jnp.full((16, 1), r, jnp.int32)
    return lax.gather(v, idx, _GDN, (1,),
                      mode=lax.GatherScatterMode.PROMISE_IN_BOUNDS)


def _sc_gather(ctable, optab, f):
    mesh = plsc.VectorSubcoreMesh(core_axis_name="c", subcore_axis_name="s",
                                  num_cores=NC, num_subcores=NS)

    TBLC = 502 * E
    CHC = 384
    NPC = R3_W // CHC
    CH3 = 768
    NP3 = R3_W // CH3

    @functools.partial(
        pl.kernel,
        out_type=[
            jax.ShapeDtypeStruct((ROWS3 * E,), jnp.float32),
            jax.ShapeDtypeStruct((ROWS3 * 16,), jnp.float32),
        ],
        mesh=mesh,
        scratch_types=[
            pltpu.VMEM((B // NW, 16), jnp.float32),
            pltpu.VMEM((TBLC,), jnp.float32),
            pltpu.VMEM((256,), jnp.float32),
            pltpu.VMEM((2, CHC * E), jnp.float32),
            pltpu.VMEM((2, CH3 * 16), jnp.float32),
            pltpu.SemaphoreType.DMA,
            pltpu.SemaphoreType.DMA,
        ],
        compiler_params=pltpu.CompilerParams(use_tc_tiling_on_sc=False,
                                             needs_layout_passes=False),
    )
    def k(ctable_hbm, f_hbm, optab_hbm, g_hbm, g2_hbm,
          colsf_v, tbl_v, otbl_v, bufc, buf3, ssem0, ssem1):
        wid = lax.axis_index("s") * NC + lax.axis_index("c")
        pltpu.sync_copy(ctable_hbm, tbl_v)
        pltpu.sync_copy(optab_hbm, otbl_v)
        pltpu.sync_copy(f_hbm.at[pl.ds(wid * (B // NW), B // NW), pl.ds(0, 16)],
                        colsf_v)

        lane = lax.iota(jnp.int32, 16)

        def ids_at(kb, col_off):
            kv = kb + lane
            bq = kv // 3
            cl = col_off + (kv - bq * 3) * 3
            return plsc.load_gather(colsf_v, [bq, cl]).astype(jnp.int32)
        ssems = (ssem0, ssem1)
        scat = [None, None]
        for p in range(NPC):
            slot = p % 2
            if scat[slot] is not None:
                scat[slot].wait()

            def grpc(g, carry, _p=p, _slot=slot):
                v = ids_at(_p * CHC + g * 16, 1)
                dst = bufc.at[_slot]
                gbase = g * (16 * E)
                for r in range(16):
                    base = _bcast_lane(v, r) * E + lane
                    for kk in range(E // 16):
                        x = plsc.load_gather(tbl_v, [base + kk * 16])
                        dst[pl.ds(gbase + r * E + kk * 16, 16)] = x
                return carry

            plsc.parallel_loop(0, CHC // 16, 1, carry=jnp.int32(0))(grpc)
            scat[slot] = pltpu.async_copy(
                bufc.at[slot],
                g_hbm.at[pl.ds((wid * R3_W + p * CHC) * E, CHC * E)],
                ssems[slot])

        scat3 = [None, None]
        for p in range(NP3):
            slot = p % 2

            def grp3(g, carry, _p=p, _slot=slot):
                v = ids_at(_p * CH3 + g * 16, 2)
                dst = buf3.at[_slot]
                gbase = g * (16 * 16)
                for r in range(16):
                    base = _bcast_lane(v, r) * 16 + lane
                    x = plsc.load_gather(otbl_v, [base])
                    dst[pl.ds(gbase + r * 16, 16)] = x
                return carry

            plsc.parallel_loop(0, CH3 // 16, 1, carry=jnp.int32(0))(grp3)
            scat3[slot] = pltpu.async_copy(
                buf3.at[slot],
                g2_hbm.at[pl.ds((wid * R3_W + p * CH3) * 16, CH3 * 16)],
                ssems[slot])
        for h in scat + scat3:
            if h is not None:
                h.wait()

    return k(ctable.reshape(-1), f, optab.reshape(-1))


def _lrelu(x):
    return jnp.where(x >= 0, x, 0.01 * x)


def _dense_body(inp, g, g2, opemb, tabemb, wf1c, wf1o, wlit, bf1, wf2, bf2,
                whr, bh, ws, bs, wfin, bfin, out):
    f32 = jnp.float32
    dot = functools.partial(jnp.dot, preferred_element_type=f32)

    tid = inp[:, 0:1].astype(jnp.int32)
    oh_t = (lax.broadcasted_iota(jnp.int32, (RBLK, 48), 1) == tid).astype(f32)
    typeg = dot(oh_t, opemb[...])
    bid = inp[:, 163:164].astype(jnp.int32)
    oh_b = (lax.broadcasted_iota(jnp.int32, (RBLK, 208), 1) == bid).astype(f32)
    tabg = dot(oh_b, tabemb[...])

    m0 = inp[:, 10:11]
    m1 = inp[:, 11:12]
    m2 = inp[:, 12:13]
    ind = [(m0 != 0).astype(f32), (m1 != 0).astype(f32), (m2 != 0).astype(f32)]
    rnf = 1.0 / (m0 + m1 + m2)

    facc = jnp.zeros((RBLK, FDIM), f32)
    for m in range(MF):
        colg = g[:, m * E:(m + 1) * E]
        opg = g2[:, 16 * m:16 * (m + 1)]
        lit = inp[:, 3 + 3 * m:4 + 3 * m]
        h = dot(colg, wf1c[...]) + dot(opg, wf1o[...]) + lit * wlit[...] + bf1[...]
        h = _lrelu(h)
        h = _lrelu(dot(h, wf2[...]) + bf2[...])
        facc = facc + ind[m] * h
    filter_emb = facc * rnf

    lane_m = lax.broadcasted_iota(jnp.int32, (RBLK, BINS * MF), 1) % MF
    tmask = jnp.where(lane_m == 0, ind[0], jnp.where(lane_m == 1, ind[1], ind[2]))
    hist_emb = dot(inp[:, 13:163] * tmask, whr[...]) * rnf + bh[...]

    tabsum = tabg + dot(inp[:, 164:1164], ws[...]) + bs[...]

    cat = jnp.concatenate([typeg, filter_emb, tabsum, hist_emb], axis=1)
    acc = dot(cat, wfin[...]) + bfin[...]
    out[...] = _lrelu(acc)


def _dense(f, g, g2, *weights):
    full = lambda shape: pl.BlockSpec(shape, lambda i: (0, 0))
    in_specs = [
        pl.BlockSpec((RBLK, D), lambda i: (i, 0)),
        pl.BlockSpec((RBLK, 3 * E), lambda i: (i, 0)),
        pl.BlockSpec((RBLK, 48), lambda i: (i, 0)),
    ] + [full(w.shape) for w in weights]
    kfn = lambda inp, gg, gg2, *ws_refs: _dense_body(
        inp[...], gg[...], gg2[...], *ws_refs[:-1], ws_refs[-1])
    return pl.pallas_call(
        kfn,
        grid=(B // RBLK,),
        in_specs=in_specs,
        out_specs=pl.BlockSpec((RBLK, FIN), lambda i: (i, 0)),
        out_shape=jax.ShapeDtypeStruct((B, FIN), jnp.float32),
        compiler_params=pltpu.CompilerParams(
            dimension_semantics=("parallel",)),
    )(f, g, g2, *weights)


def kernel(input_feature, op_emb, tab_emb, col_emb, filtop_emb,
           Wf1, bf1, Wf2, bf2, Ws, bs, Wh, bh, Wfin, bfin):
    f = input_feature
    optab = jnp.pad(filtop_emb, ((0, 2), (0, 8)))

    g, g2 = _sc_gather(col_emb, optab, f)
    g = g.reshape(B, 3 * E)
    g2 = g2.reshape(B, 48)

    wf1c = Wf1[0:E]
    wf1o = jnp.pad(Wf1[E:E + 8], ((0, 8), (0, 0)))
    wlit = Wf1[E + 8:E + 9]
    whr = jnp.repeat(Wh, MF, axis=0)
    opemb = jnp.pad(op_emb, ((0, 6), (0, 0)))
    tabemb = jnp.pad(tab_emb, ((0, 6), (0, 0)))
    weights = (opemb, tabemb,
               wf1c, wf1o, wlit, bf1.reshape(1, -1), Wf2, bf2.reshape(1, -1),
               whr, bh.reshape(1, -1), Ws, bs.reshape(1, -1),
               Wfin, bfin.reshape(1, -1))
    return _dense(f, g, g2, *weights)


# speedup vs baseline: 1.5790x; 1.5790x over previous
import functools

import jax
import jax.numpy as jnp
from jax import lax
from jax.experimental import pallas as pl
from jax.experimental.pallas import tpu as pltpu
from jax.experimental.pallas import tpu_sc as plsc

B = 16384
E = 64
MF = 3
BINS = 50
FDIM = 73
FIN = 265
D = 1164

NC, NS = 2, 16
NW = NC * NS
CHUNK = 128

ROWS3 = 3 * B
R3_W = ROWS3 // NW

RBLK = 1024


_GDN = lax.GatherDimensionNumbers(
    offset_dims=(), collapsed_slice_dims=(0,), start_index_map=(0,))


def _bcast_lane(v, r):
    idx = jnp.full((16, 1), r, jnp.int32)
    return lax.gather(v, idx, _GDN, (1,),
                      mode=lax.GatherScatterMode.PROMISE_IN_BOUNDS)


def _sc_gather(ctable, optab, f):
    mesh = plsc.VectorSubcoreMesh(core_axis_name="c", subcore_axis_name="s",
                                  num_cores=NC, num_subcores=NS)

    TBLC = 502 * E
    CHC = 384
    NPC = R3_W // CHC
    CH3 = 768
    NP3 = R3_W // CH3

    @functools.partial(
        pl.kernel,
        out_type=[
            jax.ShapeDtypeStruct((ROWS3 * E,), jnp.float32),
            jax.ShapeDtypeStruct((ROWS3 * 16,), jnp.float32),
        ],
        mesh=mesh,
        scratch_types=[
            pltpu.VMEM((B // NW * 9,), jnp.float32),
            pltpu.VMEM((TBLC,), jnp.float32),
            pltpu.VMEM((256,), jnp.float32),
            pltpu.VMEM((2, CHC * E), jnp.float32),
            pltpu.VMEM((2, CH3 * 16), jnp.float32),
            pltpu.SemaphoreType.DMA,
            pltpu.SemaphoreType.DMA,
        ],
        compiler_params=pltpu.CompilerParams(use_tc_tiling_on_sc=False,
                                             needs_layout_passes=False),
    )
    def k(ctable_hbm, colsf_hbm, optab_hbm, g_hbm, g2_hbm,
          colsf_v, tbl_v, otbl_v, bufc, buf3, ssem0, ssem1):
        wid = lax.axis_index("s") * NC + lax.axis_index("c")
        pltpu.sync_copy(ctable_hbm, tbl_v)
        pltpu.sync_copy(optab_hbm, otbl_v)
        pltpu.sync_copy(colsf_hbm.at[wid], colsf_v)

        lane = lax.iota(jnp.int32, 16)

        def ids_at(kb, off):
            return plsc.load_gather(
                colsf_v, [(kb + lane) * 3 + off]).astype(jnp.int32)
        ssems = (ssem0, ssem1)
        scat = [None, None]
        for p in range(NPC):
            slot = p % 2
            if scat[slot] is not None:
                scat[slot].wait()

            def grpc(g, carry, _p=p, _slot=slot):
                v = ids_at(_p * CHC + g * 16, 0)
                dst = bufc.at[_slot]
                gbase = g * (16 * E)
                for r in range(16):
                    base = _bcast_lane(v, r) * E + lane
                    for kk in range(E // 16):
                        x = plsc.load_gather(tbl_v, [base + kk * 16])
                        dst[pl.ds(gbase + r * E + kk * 16, 16)] = x
                return carry

            plsc.parallel_loop(0, CHC // 16, 1, carry=jnp.int32(0))(grpc)
            scat[slot] = pltpu.async_copy(
                bufc.at[slot],
                g_hbm.at[pl.ds((wid * R3_W + p * CHC) * E, CHC * E)],
                ssems[slot])

        scat3 = [None, None]
        for p in range(NP3):
            slot = p % 2

            def grp3(g, carry, _p=p, _slot=slot):
                v = ids_at(_p * CH3 + g * 16, 1)
                dst = buf3.at[_slot]
                gbase = g * (16 * 16)
                for r in range(16):
                    base = _bcast_lane(v, r) * 16 + lane
                    x = plsc.load_gather(otbl_v, [base])
                    dst[pl.ds(gbase + r * 16, 16)] = x
                return carry

            plsc.parallel_loop(0, CH3 // 16, 1, carry=jnp.int32(0))(grp3)
            scat3[slot] = pltpu.async_copy(
                buf3.at[slot],
                g2_hbm.at[pl.ds((wid * R3_W + p * CH3) * 16, CH3 * 16)],
                ssems[slot])
        for h in scat + scat3:
            if h is not None:
                h.wait()

    return k(ctable.reshape(-1), f[:, 1:10].reshape(NW, -1),
             optab.reshape(-1))


def _lrelu(x):
    return jnp.where(x >= 0, x, 0.01 * x)


def _dense_body(inp, g, g2, opemb, tabemb, wf1c, wf1o, wlit, bf1, wf2, bf2,
                whr, bh, ws, bs, wfin, bfin, out):
    f32 = jnp.float32
    dot = functools.partial(jnp.dot, preferred_element_type=f32)

    tid = inp[:, 0:1].astype(jnp.int32)
    oh_t = (lax.broadcasted_iota(jnp.int32, (RBLK, 48), 1) == tid).astype(f32)
    typeg = dot(oh_t, opemb[...])
    bid = inp[:, 163:164].astype(jnp.int32)
    oh_b = (lax.broadcasted_iota(jnp.int32, (RBLK, 208), 1) == bid).astype(f32)
    tabg = dot(oh_b, tabemb[...])

    m0 = inp[:, 10:11]
    m1 = inp[:, 11:12]
    m2 = inp[:, 12:13]
    ind = [(m0 != 0).astype(f32), (m1 != 0).astype(f32), (m2 != 0).astype(f32)]
    rnf = 1.0 / (m0 + m1 + m2)

    facc = jnp.zeros((RBLK, FDIM), f32)
    for m in range(MF):
        colg = g[:, m * E:(m + 1) * E]
        opg = g2[:, 16 * m:16 * (m + 1)]
        lit = inp[:, 3 + 3 * m:4 + 3 * m]
        h = dot(colg, wf1c[...]) + dot(opg, wf1o[...]) + lit * wlit[...] + bf1[...]
        h = _lrelu(h)
        h = _lrelu(dot(h, wf2[...]) + bf2[...])
        facc = facc + ind[m] * h
    filter_emb = facc * rnf

    lane_m = lax.broadcasted_iota(jnp.int32, (RBLK, BINS * MF), 1) % MF
    tmask = jnp.where(lane_m == 0, ind[0], jnp.where(lane_m == 1, ind[1], ind[2]))
    hist_emb = dot(inp[:, 13:163] * tmask, whr[...]) * rnf + bh[...]

    tabsum = tabg + dot(inp[:, 164:1164], ws[...]) + bs[...]

    cat = jnp.concatenate([typeg, filter_emb, tabsum, hist_emb], axis=1)
    acc = dot(cat, wfin[...]) + bfin[...]
    out[...] = _lrelu(acc)


def _dense(f, g, g2, *weights):
    full = lambda shape: pl.BlockSpec(shape, lambda i: (0, 0))
    in_specs = [
        pl.BlockSpec((RBLK, D), lambda i: (i, 0)),
        pl.BlockSpec((RBLK, 3 * E), lambda i: (i, 0)),
        pl.BlockSpec((RBLK, 48), lambda i: (i, 0)),
    ] + [full(w.shape) for w in weights]
    kfn = lambda inp, gg, gg2, *ws_refs: _dense_body(
        inp[...], gg[...], gg2[...], *ws_refs[:-1], ws_refs[-1])
    return pl.pallas_call(
        kfn,
        grid=(B // RBLK,),
        in_specs=in_specs,
        out_specs=pl.BlockSpec((RBLK, FIN), lambda i: (i, 0)),
        out_shape=jax.ShapeDtypeStruct((B, FIN), jnp.float32),
        compiler_params=pltpu.CompilerParams(
            dimension_semantics=("parallel",)),
    )(f, g, g2, *weights)


def kernel(input_feature, op_emb, tab_emb, col_emb, filtop_emb,
           Wf1, bf1, Wf2, bf2, Ws, bs, Wh, bh, Wfin, bfin):
    f = input_feature
    optab = jnp.pad(filtop_emb, ((0, 2), (0, 8)))

    g, g2 = _sc_gather(col_emb, optab, f)
    g = g.reshape(B, 3 * E)
    g2 = g2.reshape(B, 48)

    wf1c = Wf1[0:E]
    wf1o = jnp.pad(Wf1[E:E + 8], ((0, 8), (0, 0)))
    wlit = Wf1[E + 8:E + 9]
    whr = jnp.repeat(Wh, MF, axis=0)
    opemb = jnp.pad(op_emb, ((0, 6), (0, 0)))
    tabemb = jnp.pad(tab_emb, ((0, 6), (0, 0)))
    weights = (opemb, tabemb,
               wf1c, wf1o, wlit, bf1.reshape(1, -1), Wf2, bf2.reshape(1, -1),
               whr, bh.reshape(1, -1), Ws, bs.reshape(1, -1),
               Wfin, bfin.reshape(1, -1))
    return _dense(f, g, g2, *weights)
